# baseline (speedup 1.0000x reference)
import functools

import jax
import jax.numpy as jnp
from jax import lax
from jax.experimental import pallas as pl
from jax.experimental.pallas import tpu as pltpu
from jax.experimental.pallas import tpu_sc as plsc

TOPN = 64
N_NODES = 256
N_GRAPHS = 512


def _score_body(a_ref, x_ref, eye_ref, idx_ref, an_ref):
    A = a_ref[0]
    X = x_ref[0]
    eye = eye_ref[...]
    n = N_NODES
    hp = jax.lax.Precision.HIGHEST
    b = pl.program_id(0)

    colsum = jnp.sum(A, axis=0, keepdims=True)
    s = (colsum ** (-1.0)).reshape(n, 1)

    s_b = s.astype(jnp.bfloat16).astype(jnp.float32)
    A_b = A.astype(jnp.bfloat16).astype(jnp.float32)
    W = eye - s_b * A_b
    M = jnp.dot(W, X, preferred_element_type=jnp.float32)

    cur = jnp.abs(M)
    k = 1
    while k < n:
        cur = cur + jnp.roll(cur, -k, axis=1)
        k *= 2
    p_col = cur[:, 0:1]

    p_row = lax.dot_general(
        p_col, eye, (((0,), (0,)), ((), ())),
        precision=hp, preferred_element_type=jnp.float32,
    )

    ii = lax.broadcasted_iota(jnp.int32, (n, n), 0)
    jj = lax.broadcasted_iota(jnp.int32, (n, n), 1)
    before = (p_row < p_col) | ((p_row == p_col) & (jj < ii))
    rank = jnp.sum(before.astype(jnp.int32), axis=1)

    r_rows = lax.broadcasted_iota(jnp.int32, (TOPN, n), 0)
    i_cols = lax.broadcasted_iota(jnp.int32, (TOPN, n), 1)
    sel = rank[None, :] == r_rows
    S = sel.astype(jnp.float32)
    r_cols = lax.broadcasted_iota(jnp.int32, (n, TOPN), 1)
    ST = (rank[:, None] == r_cols).astype(jnp.float32)
    SA = jnp.dot(S, A, precision=hp, preferred_element_type=jnp.float32)
    an_ref[0] = jnp.dot(SA, ST, precision=hp,
                        preferred_element_type=jnp.float32)

    idx = jnp.sum(jnp.where(sel, i_cols, 0), axis=1)
    idx_ref[0, 0] = idx + b * n


def _scores_pass(A, X):
    n = N_NODES
    eye = jnp.eye(n, dtype=jnp.float32)
    fidx, A_next = pl.pallas_call(
        _score_body,
        grid=(N_GRAPHS,),
        in_specs=[
            pl.BlockSpec((1, n, n), lambda b: (b, 0, 0)),
            pl.BlockSpec((1, n, n), lambda b: (b, 0, 0)),
            pl.BlockSpec((n, n), lambda b: (0, 0)),
        ],
        out_specs=[
            pl.BlockSpec((1, 1, TOPN), lambda b: (b, 0, 0)),
            pl.BlockSpec((1, TOPN, TOPN), lambda b: (b, 0, 0)),
        ],
        out_shape=[
            jax.ShapeDtypeStruct((N_GRAPHS, 1, TOPN), jnp.int32),
            jax.ShapeDtypeStruct((N_GRAPHS, TOPN, TOPN), jnp.float32),
        ],
        compiler_params=pltpu.CompilerParams(
            dimension_semantics=("arbitrary",),
        ),
    )(A, X, eye)
    return fidx.reshape(N_GRAPHS * TOPN), A_next


def _gather_kernel(xf_hbm, fidx_hbm, h_hbm, idx_v, rows0_v, rows1_v,
                   sem0, sem1):
    g_per_w = N_GRAPHS // 32
    nc = 2
    wid = lax.axis_index("s") * nc + lax.axis_index("c")
    g0 = wid * g_per_w
    pltpu.sync_copy(fidx_hbm.at[pl.ds(g0 * TOPN, g_per_w * TOPN)], idx_v)

    bufs = (rows0_v, rows1_v)
    sems = (sem0, sem1)

    rows_per_chunk = 2 * TOPN
    n_chunks = g_per_w // 2

    def issue(ci, buf, sem):
        idx_slice = idx_v.at[pl.ds(ci * rows_per_chunk, rows_per_chunk)]
        return pltpu.async_copy(xf_hbm.at[idx_slice], buf, sem)

    issue(0, bufs[0], sems[0])
    for ci in range(n_chunks):
        cur = bufs[ci % 2]
        if ci + 1 < n_chunks:
            issue(ci + 1, bufs[(ci + 1) % 2], sems[(ci + 1) % 2])
        pltpu.make_async_copy(xf_hbm.at[pl.ds(0, rows_per_chunk)],
                              cur, sems[ci % 2]).wait()
        row0 = (g0 * TOPN) + ci * rows_per_chunk
        pltpu.sync_copy(cur, h_hbm.at[pl.ds(row0, rows_per_chunk)])


def _sc_gather(Xf, fidx):
    n = N_NODES
    rows_out = N_GRAPHS * TOPN
    g_per_w = N_GRAPHS // 32
    mesh = plsc.VectorSubcoreMesh(core_axis_name="c", subcore_axis_name="s")
    k = functools.partial(
        pl.kernel,
        mesh=mesh,
        out_type=jax.ShapeDtypeStruct((rows_out, n), jnp.float32),
        scratch_types=[
            pltpu.VMEM((g_per_w * TOPN,), jnp.int32),
            pltpu.VMEM((2 * TOPN, n), jnp.float32),
            pltpu.VMEM((2 * TOPN, n), jnp.float32),
            pltpu.SemaphoreType.DMA,
            pltpu.SemaphoreType.DMA,
        ],
    )(_gather_kernel)
    return k(Xf, fidx)


def kernel(A, X):
    n = N_NODES
    fidx, A_next = _scores_pass(A, X)
    Xf = X.reshape(N_GRAPHS * n, n)
    Hf = _sc_gather(Xf, fidx)
    H = Hf.reshape(N_GRAPHS, TOPN, n)
    return (A_next, H)
